# baseline (speedup 1.0000x reference)
import jax
import jax.numpy as jnp
from jax.experimental import pallas as pl
from jax.experimental.pallas import tpu as pltpu

_D_MODEL = 2048
_N_EXP = 16
_T = 16384
_TM = 1024


def _router_block(h_ref, wt_ref, b_ref, mask_ref, probs_ref, logits_ref):
    logits = jnp.dot(h_ref[...], wt_ref[...], preferred_element_type=jnp.float32)
    logits = logits + b_ref[...]
    logits_ref[...] = logits

    m1 = jnp.max(logits, axis=-1, keepdims=True)
    e = jnp.exp(logits - m1)
    probs_ref[...] = e / jnp.sum(e, axis=-1, keepdims=True)

    col = jax.lax.broadcasted_iota(jnp.int32, logits.shape, 1)
    i1 = jnp.min(jnp.where(logits == m1, col, _N_EXP), axis=-1, keepdims=True)
    rest = jnp.where(col == i1, -jnp.inf, logits)
    m2 = jnp.max(rest, axis=-1, keepdims=True)
    i2 = jnp.min(jnp.where(rest == m2, col, _N_EXP), axis=-1, keepdims=True)
    mask_ref[...] = ((col == i1) | (col == i2)).astype(jnp.float32)


def kernel(h, bias_row, W):
    wt = W.T
    b = jnp.zeros((1, _N_EXP), jnp.float32).at[0, _N_EXP - 1].set(bias_row[-1])
    grid = (_T // _TM,)
    out_shapes = (
        jax.ShapeDtypeStruct((_T, _N_EXP), jnp.float32),
        jax.ShapeDtypeStruct((_T, _N_EXP), jnp.float32),
        jax.ShapeDtypeStruct((_T, _N_EXP), jnp.float32),
    )
    out_spec = pl.BlockSpec((_TM, _N_EXP), lambda i: (i, 0))
    mask_f, probs, logits = pl.pallas_call(
        _router_block,
        grid=grid,
        in_specs=[
            pl.BlockSpec((_TM, _D_MODEL), lambda i: (i, 0)),
            pl.BlockSpec((_D_MODEL, _N_EXP), lambda i: (0, 0)),
            pl.BlockSpec((1, _N_EXP), lambda i: (0, 0)),
        ],
        out_specs=(out_spec, out_spec, out_spec),
        out_shape=out_shapes,
        compiler_params=pltpu.CompilerParams(
            dimension_semantics=("arbitrary",),
        ),
    )(h, wt, b)
    return (mask_f.astype(bool), probs, logits)


# speedup vs baseline: 1.0406x; 1.0406x over previous
import jax
import jax.numpy as jnp
from jax.experimental import pallas as pl
from jax.experimental.pallas import tpu as pltpu

_D_MODEL = 2048
_N_EXP = 16
_T = 16384
_TM = 2048


def _router_block(h_ref, wt_ref, b_ref, mask_ref, probs_ref, logits_ref):
    logits = jnp.dot(h_ref[...], wt_ref[...], preferred_element_type=jnp.float32)
    logits = logits + b_ref[...]
    logits_ref[...] = logits

    m1 = jnp.max(logits, axis=-1, keepdims=True)
    e = jnp.exp(logits - m1)
    probs_ref[...] = e / jnp.sum(e, axis=-1, keepdims=True)

    col = jax.lax.broadcasted_iota(jnp.int32, logits.shape, 1)
    i1 = jnp.min(jnp.where(logits == m1, col, _N_EXP), axis=-1, keepdims=True)
    rest = jnp.where(col == i1, -jnp.inf, logits)
    m2 = jnp.max(rest, axis=-1, keepdims=True)
    i2 = jnp.min(jnp.where(rest == m2, col, _N_EXP), axis=-1, keepdims=True)
    mask_ref[...] = ((col == i1) | (col == i2)).astype(jnp.float32)


def kernel(h, bias_row, W):
    wt = W.T
    b = jnp.zeros((1, _N_EXP), jnp.float32).at[0, _N_EXP - 1].set(bias_row[-1])
    grid = (_T // _TM,)
    out_shapes = (
        jax.ShapeDtypeStruct((_T, _N_EXP), jnp.float32),
        jax.ShapeDtypeStruct((_T, _N_EXP), jnp.float32),
        jax.ShapeDtypeStruct((_T, _N_EXP), jnp.float32),
    )
    out_spec = pl.BlockSpec((_TM, _N_EXP), lambda i: (i, 0))
    mask_f, probs, logits = pl.pallas_call(
        _router_block,
        grid=grid,
        in_specs=[
            pl.BlockSpec((_TM, _D_MODEL), lambda i: (i, 0)),
            pl.BlockSpec((_D_MODEL, _N_EXP), lambda i: (0, 0)),
            pl.BlockSpec((1, _N_EXP), lambda i: (0, 0)),
        ],
        out_specs=(out_spec, out_spec, out_spec),
        out_shape=out_shapes,
        compiler_params=pltpu.CompilerParams(
            dimension_semantics=("arbitrary",),
        ),
    )(h, wt, b)
    return (mask_f.astype(bool), probs, logits)
